# Initial kernel scaffold; baseline (speedup 1.0000x reference)
import jax
import jax.numpy as jnp
from jax.experimental import pallas as pl


def kernel(x, forward_order, band_indices, rho_bar):
    raise NotImplementedError("write your pallas kernel here")


# speedup vs baseline: 6.5445x; 6.5445x over previous
import functools

import jax
import jax.numpy as jnp
from jax import lax
from jax.experimental import pallas as pl
from jax.experimental.pallas import tpu as pltpu
from jax.experimental.pallas import tpu_sc as plsc

_NUM_CORES = 2
_NUM_SUBCORES = 16
_NW = _NUM_CORES * _NUM_SUBCORES

_CHUNK = 128


def _make_gather(total_rows: int, d: int):
    rows_per_worker = total_rows // _NW
    n_chunks = rows_per_worker // _CHUNK
    mesh = plsc.VectorSubcoreMesh(
        core_axis_name="c",
        subcore_axis_name="s",
        num_cores=_NUM_CORES,
        num_subcores=_NUM_SUBCORES,
    )

    @functools.partial(
        pl.kernel,
        mesh=mesh,
        out_type=jax.ShapeDtypeStruct((total_rows, d), jnp.float32),
        scratch_types=[
            pltpu.VMEM((n_chunks, _CHUNK), jnp.int32),
            pltpu.VMEM((_CHUNK, d), jnp.float32),
            pltpu.SemaphoreType.DMA,
        ],
    )
    def gather(table_hbm, idx_hbm, out_hbm, idx_v, buf, sem):
        wid = lax.axis_index("s") * _NUM_CORES + lax.axis_index("c")
        base = wid * rows_per_worker
        pltpu.sync_copy(idx_hbm.at[wid], idx_v)

        @pl.loop(0, n_chunks)
        def _(g):
            pltpu.async_copy(table_hbm.at[idx_v.at[g]], buf, sem).wait()
            pltpu.sync_copy(buf, out_hbm.at[pl.ds(base + g * _CHUNK, _CHUNK)])

    return gather


def kernel(x, forward_order, band_indices, rho_bar):
    b, n, d = x.shape
    total = b * n
    table = x.reshape(total, d)
    offs = (jnp.arange(b, dtype=jnp.int32) * n)[:, None]
    idx = (forward_order.astype(jnp.int32)[None, :] + offs).reshape(
        _NW, total // (_NW * _CHUNK), _CHUNK
    )
    out = _make_gather(total, d)(table, idx)
    return (out.reshape(b, n, d), band_indices, rho_bar)


# speedup vs baseline: 7.5485x; 1.1534x over previous
import functools

import jax
import jax.numpy as jnp
from jax import lax
from jax.experimental import pallas as pl
from jax.experimental.pallas import tpu as pltpu
from jax.experimental.pallas import tpu_sc as plsc

_NUM_CORES = 2
_NUM_SUBCORES = 16
_NW = _NUM_CORES * _NUM_SUBCORES

_CHUNK = 128


def _make_gather(total_rows: int, d: int):
    rows_per_worker = total_rows // _NW
    n_chunks = rows_per_worker // _CHUNK
    mesh = plsc.VectorSubcoreMesh(
        core_axis_name="c",
        subcore_axis_name="s",
        num_cores=_NUM_CORES,
        num_subcores=_NUM_SUBCORES,
    )

    @functools.partial(
        pl.kernel,
        mesh=mesh,
        out_type=jax.ShapeDtypeStruct((total_rows, d), jnp.float32),
        scratch_types=[
            pltpu.VMEM((n_chunks, _CHUNK), jnp.int32),
            pltpu.VMEM((_CHUNK, d), jnp.float32),
            pltpu.VMEM((_CHUNK, d), jnp.float32),
            pltpu.SemaphoreType.DMA,
            pltpu.SemaphoreType.DMA,
        ],
    )
    def gather(table_hbm, idx_hbm, out_hbm, idx_v, buf0, buf1, sem0, sem1):
        wid = lax.axis_index("s") * _NUM_CORES + lax.axis_index("c")
        base = wid * rows_per_worker
        pltpu.sync_copy(idx_hbm.at[wid], idx_v)
        pltpu.async_copy(table_hbm.at[idx_v.at[0]], buf0, sem0)

        @pl.loop(0, n_chunks, step=2)
        def _(g):
            pltpu.make_async_copy(table_hbm.at[idx_v.at[g]], buf0, sem0).wait()
            pltpu.async_copy(table_hbm.at[idx_v.at[g + 1]], buf1, sem1)
            pltpu.sync_copy(buf0, out_hbm.at[pl.ds(base + g * _CHUNK, _CHUNK)])
            pltpu.make_async_copy(
                table_hbm.at[idx_v.at[g + 1]], buf1, sem1
            ).wait()

            @pl.when(g + 2 < n_chunks)
            def _():
                pltpu.async_copy(table_hbm.at[idx_v.at[g + 2]], buf0, sem0)

            pltpu.sync_copy(
                buf1, out_hbm.at[pl.ds(base + (g + 1) * _CHUNK, _CHUNK)]
            )

    return gather


def kernel(x, forward_order, band_indices, rho_bar):
    b, n, d = x.shape
    total = b * n
    table = x.reshape(total, d)
    offs = (jnp.arange(b, dtype=jnp.int32) * n)[:, None]
    idx = (forward_order.astype(jnp.int32)[None, :] + offs).reshape(
        _NW, total // (_NW * _CHUNK), _CHUNK
    )
    out = _make_gather(total, d)(table, idx)
    return (out.reshape(b, n, d), band_indices, rho_bar)


# speedup vs baseline: 7.7316x; 1.0243x over previous
import functools

import jax
import jax.numpy as jnp
from jax import lax
from jax.experimental import pallas as pl
from jax.experimental.pallas import tpu as pltpu
from jax.experimental.pallas import tpu_sc as plsc

_NUM_CORES = 2
_NUM_SUBCORES = 16
_NW = _NUM_CORES * _NUM_SUBCORES

_CHUNK = 64
_NBUF = 4


def _make_gather(total_rows: int, d: int):
    rows_per_worker = total_rows // _NW
    n_chunks = rows_per_worker // _CHUNK
    assert n_chunks % _NBUF == 0
    mesh = plsc.VectorSubcoreMesh(
        core_axis_name="c",
        subcore_axis_name="s",
        num_cores=_NUM_CORES,
        num_subcores=_NUM_SUBCORES,
    )

    @functools.partial(
        pl.kernel,
        mesh=mesh,
        out_type=jax.ShapeDtypeStruct((total_rows, d), jnp.float32),
        scratch_types=[
            pltpu.VMEM((n_chunks, _CHUNK), jnp.int32),
            *[pltpu.VMEM((_CHUNK, d), jnp.float32) for _ in range(_NBUF)],
            *[pltpu.SemaphoreType.DMA for _ in range(2 * _NBUF)],
        ],
    )
    def gather(table_hbm, idx_hbm, out_hbm, idx_v, *rest):
        bufs = rest[:_NBUF]
        gsem = rest[_NBUF : 2 * _NBUF]
        ssem = rest[2 * _NBUF :]
        wid = lax.axis_index("s") * _NUM_CORES + lax.axis_index("c")
        base = wid * rows_per_worker

        def fire_gather(c, j):
            pltpu.async_copy(table_hbm.at[idx_v.at[c]], bufs[j], gsem[j])

        def wait_gather(c, j):
            pltpu.make_async_copy(
                table_hbm.at[idx_v.at[c]], bufs[j], gsem[j]
            ).wait()

        def out_slice(c):
            return out_hbm.at[pl.ds(base + c * _CHUNK, _CHUNK)]

        def fire_store(c, j):
            pltpu.async_copy(bufs[j], out_slice(c), ssem[j])

        def wait_store(c, j):
            pltpu.make_async_copy(bufs[j], out_slice(c), ssem[j]).wait()

        pltpu.sync_copy(idx_hbm.at[wid], idx_v)
        fire_gather(0, 0)
        fire_gather(1, 1)

        @pl.loop(0, n_chunks, step=_NBUF)
        def _(g):
            for j in range(_NBUF):
                c = g + j
                jj = (j + 2) % _NBUF

                @pl.when(c >= 2)
                def _():
                    wait_store(c - 2, jj)

                @pl.when(c + 2 < n_chunks)
                def _():
                    fire_gather(c + 2, jj)

                wait_gather(c, j)
                fire_store(c, j)

        wait_store(n_chunks - 2, (n_chunks - 2) % _NBUF)
        wait_store(n_chunks - 1, (n_chunks - 1) % _NBUF)

    return gather


def kernel(x, forward_order, band_indices, rho_bar):
    b, n, d = x.shape
    total = b * n
    table = x.reshape(total, d)
    offs = (jnp.arange(b, dtype=jnp.int32) * n)[:, None]
    idx = (forward_order.astype(jnp.int32)[None, :] + offs).reshape(
        _NW, total // (_NW * _CHUNK), _CHUNK
    )
    out = _make_gather(total, d)(table, idx)
    return (out.reshape(b, n, d), band_indices, rho_bar)


# speedup vs baseline: 7.7414x; 1.0013x over previous
import functools

import jax
import jax.numpy as jnp
from jax import lax
from jax.experimental import pallas as pl
from jax.experimental.pallas import tpu as pltpu
from jax.experimental.pallas import tpu_sc as plsc

_NUM_CORES = 2
_NUM_SUBCORES = 16
_NW = _NUM_CORES * _NUM_SUBCORES

_CHUNK = 64
_NBUF = 4


def _make_gather(b: int, n: int, d: int):
    total_rows = b * n
    rows_per_worker = total_rows // _NW
    n_chunks = rows_per_worker // _CHUNK
    workers_per_batch = n // rows_per_worker
    assert n_chunks % _NBUF == 0 and workers_per_batch * rows_per_worker == n
    mesh = plsc.VectorSubcoreMesh(
        core_axis_name="c",
        subcore_axis_name="s",
        num_cores=_NUM_CORES,
        num_subcores=_NUM_SUBCORES,
    )

    @functools.partial(
        pl.kernel,
        mesh=mesh,
        out_type=jax.ShapeDtypeStruct((b, n, d), jnp.float32),
        scratch_types=[
            pltpu.VMEM((n_chunks, _CHUNK), jnp.int32),
            *[pltpu.VMEM((_CHUNK, d), jnp.float32) for _ in range(_NBUF)],
            *[pltpu.SemaphoreType.DMA for _ in range(2 * _NBUF)],
        ],
    )
    def gather(x_hbm, fo_hbm, out_hbm, idx_v, *rest):
        bufs = rest[:_NBUF]
        gsem = rest[_NBUF : 2 * _NBUF]
        ssem = rest[2 * _NBUF :]
        wid = lax.axis_index("s") * _NUM_CORES + lax.axis_index("c")
        batch = wid // workers_per_batch
        row0 = (wid % workers_per_batch) * rows_per_worker

        def fire_gather(c, j):
            pltpu.async_copy(
                x_hbm.at[batch].at[idx_v.at[c]], bufs[j], gsem[j]
            )

        def wait_gather(c, j):
            pltpu.make_async_copy(
                x_hbm.at[batch].at[idx_v.at[c]], bufs[j], gsem[j]
            ).wait()

        def out_slice(c):
            return out_hbm.at[batch].at[pl.ds(row0 + c * _CHUNK, _CHUNK)]

        def fire_store(c, j):
            pltpu.async_copy(bufs[j], out_slice(c), ssem[j])

        def wait_store(c, j):
            pltpu.make_async_copy(bufs[j], out_slice(c), ssem[j]).wait()

        pltpu.sync_copy(
            fo_hbm.at[pl.ds((wid % workers_per_batch) * n_chunks, n_chunks)],
            idx_v,
        )
        fire_gather(0, 0)
        fire_gather(1, 1)

        @pl.loop(0, n_chunks, step=_NBUF)
        def _(g):
            for j in range(_NBUF):
                c = g + j
                jj = (j + 2) % _NBUF

                @pl.when(c >= 2)
                def _():
                    wait_store(c - 2, jj)

                @pl.when(c + 2 < n_chunks)
                def _():
                    fire_gather(c + 2, jj)

                wait_gather(c, j)
                fire_store(c, j)

        wait_store(n_chunks - 2, (n_chunks - 2) % _NBUF)
        wait_store(n_chunks - 1, (n_chunks - 1) % _NBUF)

    return gather


def kernel(x, forward_order, band_indices, rho_bar):
    b, n, d = x.shape
    fo = forward_order.astype(jnp.int32).reshape(n // _CHUNK, _CHUNK)
    out = _make_gather(b, n, d)(x, fo)
    return (out, band_indices, rho_bar)
